# baseline (speedup 1.0000x reference)
import jax
import jax.numpy as jnp
from jax import lax
from jax.experimental import pallas as pl

NG = 128
KS = 32
CP = 8


def _fps_body(x_ref, y_ref, z_ref, cx_ref, cy_ref, cz_ref):
    x, y, z = x_ref[...], y_ref[...], z_ref[...]
    b, n = x.shape
    lane = lax.broadcasted_iota(jnp.int32, (1, n), 1)
    lane_g = lax.broadcasted_iota(jnp.int32, (b, NG), 1)

    def step(i, carry):
        idxf, dists, ax, ay, az = carry
        onehot = jnp.where(lane == idxf, 1.0, 0.0)
        cx = jnp.sum(x * onehot, axis=1, keepdims=True)
        cy = jnp.sum(y * onehot, axis=1, keepdims=True)
        cz = jnp.sum(z * onehot, axis=1, keepdims=True)
        ax = jnp.where(lane_g == i, cx, ax)
        ay = jnp.where(lane_g == i, cy, ay)
        az = jnp.where(lane_g == i, cz, az)
        dx = x - cx
        dy = y - cy
        dz = z - cz
        d = (dx * dx + dy * dy) + dz * dz
        dists = jnp.minimum(dists, d)
        m = jnp.max(dists, axis=1, keepdims=True)
        idxf = jnp.min(jnp.where(dists == m, lane, n),
                       axis=1, keepdims=True)
        return idxf, dists, ax, ay, az

    idxf0 = jnp.zeros((b, 1), jnp.int32)
    dists0 = jnp.full((b, n), 1e10, jnp.float32)
    acc0 = jnp.zeros((b, NG), jnp.float32)
    _, _, ax, ay, az = lax.fori_loop(
        0, NG, step, (idxf0, dists0, acc0, acc0, acc0))
    cx_ref[...] = ax
    cy_ref[...] = ay
    cz_ref[...] = az


def _knn_body(pts_ref, cen_ref, cent_ref, org_ref, ctr_ref):
    pts = pts_ref[0, 0]
    cen = cen_ref[0]
    ct = cent_ref[0]
    n = pts.shape[1]

    sq = pts * pts
    p2 = (sq[0:1] + sq[1:2]) + sq[2:3]
    cs = ct * ct
    c2 = (cs[:, 0:1] + cs[:, 1:2]) + cs[:, 2:3]

    e = lax.dot_general(cen, pts, (((0,), (0,)), ((), ())),
                        preferred_element_type=jnp.float32)
    d2 = (c2 + p2) - 2.0 * e

    lane = lax.broadcasted_iota(jnp.int32, (NG, n), 1)
    slot = lax.broadcasted_iota(jnp.int32, (NG, KS * CP), 1) // CP

    def step(k, carry):
        d2, org_acc, ctr_acc = carry
        mn = jnp.min(d2, axis=1, keepdims=True)
        idxf = jnp.min(jnp.where(d2 == mn, lane, n),
                       axis=1, keepdims=True)
        self = jnp.where(lane == idxf, 1.0, 0.0)
        d2 = d2 + self * 3e38
        p = lax.dot_general(self, pts,
                            (((1,), (1,)), ((), ())),
                            preferred_element_type=jnp.float32,
                            precision=lax.Precision.HIGHEST)
        p_t = jnp.concatenate([p] * KS, axis=1)
        c_t = jnp.concatenate([p - ct] * KS, axis=1)
        org_acc = jnp.where(slot == k, p_t, org_acc)
        ctr_acc = jnp.where(slot == k, c_t, ctr_acc)
        return d2, org_acc, ctr_acc

    acc0 = jnp.zeros((NG, KS * CP), jnp.float32)
    _, org_acc, ctr_acc = lax.fori_loop(0, KS, step, (d2, acc0, acc0))
    org_ref[0] = org_acc
    ctr_ref[0] = ctr_acc


def kernel(xyz):
    V, B, N, _ = xyz.shape
    xyz_t = jnp.transpose(xyz, (0, 1, 3, 2))
    xyz_t = jnp.concatenate(
        [xyz_t, jnp.zeros((V, B, CP - 3, N), xyz.dtype)], axis=2)

    cx, cy, cz = pl.pallas_call(
        _fps_body,
        out_shape=[jax.ShapeDtypeStruct((B, NG), jnp.float32)] * 3,
    )(xyz_t[0, :, 0], xyz_t[0, :, 1], xyz_t[0, :, 2])
    centers = jnp.stack(
        [cx, cy, cz, jnp.zeros_like(cx), jnp.zeros_like(cx),
         jnp.zeros_like(cx), jnp.zeros_like(cx), jnp.zeros_like(cx)],
        axis=1)

    org, ctr = pl.pallas_call(
        _knn_body,
        grid=(B * V,),
        in_specs=[
            pl.BlockSpec((1, 1, CP, N), lambda p: (p % V, p // V, 0, 0)),
            pl.BlockSpec((1, CP, NG), lambda p: (p // V, 0, 0)),
            pl.BlockSpec((1, NG, CP), lambda p: (p // V, 0, 0)),
        ],
        out_specs=[
            pl.BlockSpec((1, NG, KS * CP), lambda p: (p, 0, 0)),
            pl.BlockSpec((1, NG, KS * CP), lambda p: (p, 0, 0)),
        ],
        out_shape=[
            jax.ShapeDtypeStruct((B * V, NG, KS * CP), jnp.float32),
            jax.ShapeDtypeStruct((B * V, NG, KS * CP), jnp.float32),
        ],
    )(xyz_t, centers, jnp.transpose(centers, (0, 2, 1)))

    neighborhood_org = org.reshape(B * V, NG, KS, CP)[..., :3]
    neighborhood = ctr.reshape(B * V, NG, KS, CP)[..., :3]
    cen3 = jnp.transpose(centers, (0, 2, 1))[..., :3]
    center_flat = jnp.broadcast_to(
        cen3[:, None], (B, V, NG, 3)).reshape(B * V, NG, 3)
    return neighborhood, center_flat, neighborhood_org


# speedup vs baseline: 3.1355x; 3.1355x over previous
import functools
import jax
import jax.numpy as jnp
from jax import lax
from jax.experimental import pallas as pl
from jax.experimental.pallas import tpu as pltpu
from jax.experimental.pallas import tpu_sc as plsc

NG = 128
KS = 32
CP = 8
NSEG = 32
GD = 16


def _fps_body(x_ref, y_ref, z_ref, cx_ref, cy_ref, cz_ref):
    x, y, z = x_ref[...], y_ref[...], z_ref[...]
    b, n = x.shape
    lane = lax.broadcasted_iota(jnp.int32, (1, n), 1)
    lane_g = lax.broadcasted_iota(jnp.int32, (b, NG), 1)

    def step(i, carry):
        idxf, dists, ax, ay, az = carry
        onehot = jnp.where(lane == idxf, 1.0, 0.0)
        cx = jnp.sum(x * onehot, axis=1, keepdims=True)
        cy = jnp.sum(y * onehot, axis=1, keepdims=True)
        cz = jnp.sum(z * onehot, axis=1, keepdims=True)
        ax = jnp.where(lane_g == i, cx, ax)
        ay = jnp.where(lane_g == i, cy, ay)
        az = jnp.where(lane_g == i, cz, az)
        dx = x - cx
        dy = y - cy
        dz = z - cz
        d = (dx * dx + dy * dy) + dz * dz
        dists = jnp.minimum(dists, d)
        m = jnp.max(dists, axis=1, keepdims=True)
        idxf = jnp.min(jnp.where(dists == m, lane, n),
                       axis=1, keepdims=True)
        return idxf, dists, ax, ay, az

    idxf0 = jnp.zeros((b, 1), jnp.int32)
    dists0 = jnp.full((b, n), 1e10, jnp.float32)
    acc0 = jnp.zeros((b, NG), jnp.float32)
    _, _, ax, ay, az = lax.fori_loop(
        0, NG, step, (idxf0, dists0, acc0, acc0, acc0))
    cx_ref[...] = ax
    cy_ref[...] = ay
    cz_ref[...] = az


def _ce(ks, xs, i, j, asc):
    a, b = ks[i], ks[j]
    c = (a <= b) if asc else (a >= b)
    ks[i] = jnp.where(c, a, b)
    ks[j] = jnp.where(c, b, a)
    ai, bi = xs[i], xs[j]
    xs[i] = jnp.where(c, ai, bi)
    xs[j] = jnp.where(c, bi, ai)


def _knn_body(pts_ref, cen_ref, cent_ref, idx_ref, *, nv, nb):
    pts = pts_ref[0, 0]
    cen = cen_ref[0]
    n = pts.shape[1]
    w = n // NSEG

    sq = pts * pts
    p2 = (sq[0:1] + sq[1:2]) + sq[2:3]
    c2 = lax.slice(cent_ref[0], (0, 3), (NG, 4))

    e = lax.dot_general(cen, pts, (((0,), (0,)), ((), ())),
                        preferred_element_type=jnp.float32)
    d2 = (c2 + p2) - 2.0 * e

    ks = [lax.slice(d2, (0, s * w), (NG, (s + 1) * w)) for s in range(NSEG)]
    base_i = lax.broadcasted_iota(jnp.int32, (NG, w), 1)
    xs = [base_i + s * w for s in range(NSEG)]

    size = 2
    while size <= NSEG:
        stride = size // 2
        while stride >= 1:
            for i in range(NSEG):
                j = i ^ stride
                if j > i:
                    _ce(ks, xs, i, j, (i & size) == 0)
            stride //= 2
        size *= 2

    while w > 1:
        h = w // 2
        nk, nx = [], []
        for i in range(NSEG):
            a = lax.slice(ks[i], (0, 0), (NG, h))
            b = lax.slice(ks[NSEG - 1 - i], (0, h), (NG, w))
            ai = lax.slice(xs[i], (0, 0), (NG, h))
            bi = lax.slice(xs[NSEG - 1 - i], (0, h), (NG, w))
            c = a <= b
            nk.append(jnp.where(c, a, b))
            nx.append(jnp.where(c, ai, bi))
        ks, xs, w = nk, nx, h
        for stride in (16, 8, 4, 2, 1):
            for i in range(NSEG):
                j = i ^ stride
                if j > i:
                    _ce(ks, xs, i, j, True)

    p = pl.program_id(0)
    base = ((p % nv) * nb + p // nv) * n
    idx_ref[0] = jnp.concatenate(xs, axis=1) + base


def _finish_body(raw_ref, cent_ref, org_ref, ctr_ref):
    raw = raw_ref[0]
    ct = cent_ref[0]
    a = lax.broadcasted_iota(jnp.int32, (KS * GD, KS * CP), 0)
    b = lax.broadcasted_iota(jnp.int32, (KS * GD, KS * CP), 1)
    sel = jnp.where((a // GD == b // CP) & (a % GD == b % CP), 1.0, 0.0)
    org = lax.dot_general(raw, sel, (((1,), (0,)), ((), ())),
                          preferred_element_type=jnp.float32,
                          precision=lax.Precision.HIGHEST)
    org_ref[0] = org
    ctr_ref[0] = org - jnp.concatenate([ct] * KS, axis=1)


def _make_gather(n_rows, table_rows):
    mesh = plsc.VectorSubcoreMesh(core_axis_name="c", subcore_axis_name="s")
    per_w = n_rows // 32
    n_ch = 2
    ch = per_w // n_ch
    jr = ch // 128

    @functools.partial(
        pl.kernel, mesh=mesh,
        compiler_params=pltpu.CompilerParams(use_tc_tiling_on_sc=False),
        out_type=jax.ShapeDtypeStruct((n_rows, GD), jnp.float32),
        scratch_types=[
            pltpu.VMEM((jr, 128), jnp.int32),
            pltpu.VMEM((ch, GD), jnp.float32),
            pltpu.SemaphoreType.DMA,
        ],
    )
    def gather(table_hbm, idx_hbm, out_hbm, idx_v, rows_v, sem):
        wid = lax.axis_index("s") * 2 + lax.axis_index("c")
        for c in range(n_ch):
            base = pl.multiple_of(wid * per_w + c * ch, 128)
            pltpu.sync_copy(
                idx_hbm.at[pl.ds(pl.multiple_of(base // 128, 8), jr)],
                idx_v)

            def body(j, carry):
                cp = pltpu.make_async_copy(
                    table_hbm.at[idx_v.at[j]],
                    rows_v.at[pl.ds(j * 128, 128)], sem)
                cp.start()
                cp.wait()
                return carry

            lax.fori_loop(0, jr, body, 0)
            pltpu.sync_copy(rows_v, out_hbm.at[pl.ds(base, ch)])
            del base

    return gather


def kernel(xyz):
    V, B, N, _ = xyz.shape
    xyz_t = jnp.transpose(xyz, (0, 1, 3, 2))
    xyz_t = jnp.concatenate(
        [xyz_t, jnp.zeros((V, B, CP - 3, N), xyz.dtype)], axis=2)

    cx, cy, cz = pl.pallas_call(
        _fps_body,
        out_shape=[jax.ShapeDtypeStruct((B, NG), jnp.float32)] * 3,
    )(xyz_t[0, :, 0], xyz_t[0, :, 1], xyz_t[0, :, 2])
    centers = jnp.stack(
        [cx, cy, cz, jnp.zeros_like(cx), jnp.zeros_like(cx),
         jnp.zeros_like(cx), jnp.zeros_like(cx), jnp.zeros_like(cx)],
        axis=1)

    c2 = (cx * cx + cy * cy) + cz * cz
    zero = jnp.zeros_like(cx)
    cen_t = jnp.stack([cx, cy, cz, c2, zero, zero, zero, zero],
                      axis=2)

    idx = pl.pallas_call(
        functools.partial(_knn_body, nv=V, nb=B),
        grid=(B * V,),
        in_specs=[
            pl.BlockSpec((1, 1, CP, N), lambda p: (p % V, p // V, 0, 0)),
            pl.BlockSpec((1, CP, NG), lambda p: (p // V, 0, 0)),
            pl.BlockSpec((1, NG, CP), lambda p: (p // V, 0, 0)),
        ],
        out_specs=pl.BlockSpec((1, NG, KS), lambda p: (p, 0, 0)),
        out_shape=jax.ShapeDtypeStruct((B * V, NG, KS), jnp.int32),
    )(xyz_t, centers, cen_t)

    n_rows = B * V * NG * KS
    table = jnp.pad(xyz.reshape(V * B * N, 3), ((0, 0), (0, GD - 3)))
    raw = _make_gather(n_rows, V * B * N)(
        table, idx.reshape(n_rows // 128, 128))

    org, ctr = pl.pallas_call(
        _finish_body,
        grid=(B * V,),
        in_specs=[
            pl.BlockSpec((1, NG, KS * GD), lambda p: (p, 0, 0)),
            pl.BlockSpec((1, NG, CP), lambda p: (p // V, 0, 0)),
        ],
        out_specs=[
            pl.BlockSpec((1, NG, KS * CP), lambda p: (p, 0, 0)),
            pl.BlockSpec((1, NG, KS * CP), lambda p: (p, 0, 0)),
        ],
        out_shape=[
            jax.ShapeDtypeStruct((B * V, NG, KS * CP), jnp.float32),
            jax.ShapeDtypeStruct((B * V, NG, KS * CP), jnp.float32),
        ],
    )(raw.reshape(B * V, NG, KS * GD), cen_t)

    neighborhood_org = org.reshape(B * V, NG, KS, CP)[..., :3]
    neighborhood = ctr.reshape(B * V, NG, KS, CP)[..., :3]
    cen3 = cen_t[..., :3]
    center_flat = jnp.broadcast_to(
        cen3[:, None], (B, V, NG, 3)).reshape(B * V, NG, 3)
    return neighborhood, center_flat, neighborhood_org
